# baseline (speedup 1.0000x reference)
import jax
import jax.numpy as jnp
from jax import lax
from jax.experimental import pallas as pl

DIM = 192
E = 8
H = 64
W = 64
HW = H * W


def _lshift(a, k):
    if k > 0:
        return jnp.concatenate(
            [a[:, k:], jnp.zeros((a.shape[0], k), a.dtype)], axis=1)
    if k < 0:
        return jnp.concatenate(
            [jnp.zeros((a.shape[0], -k), a.dtype), a[:, :k]], axis=1)
    return a


def _mm(w, a):
    return lax.dot_general(w, a, (((1,), (0,)), ((), ())),
                           preferred_element_type=jnp.float32)


def _moe_step(x_ref, rw_ref, rb_ref, eb_ref,
              w0_ref, b0_ref, dw0_ref, db0_ref,
              w1_ref, b1_ref, g1_ref,
              w2_ref, b2_ref,
              f3a_ref, f3ab_ref, f3b_ref, f3bb_ref,
              w4_ref, b4_ref, dw4_ref, db4_ref,
              w5_ref, b5_ref, g5_ref,
              w6_ref, b6_ref,
              f7a_ref, f7ab_ref, f7b_ref, f7bb_ref,
              out_ref):
    xb = x_ref[0]

    gvec = jnp.mean(xb, axis=1, keepdims=True)
    logits = _mm(rw_ref[...], gvec) + rb_ref[...]
    logits = jnp.clip(logits, -10.0, 10.0) + eb_ref[...]
    m = jnp.max(logits)
    p = jnp.exp(logits - m)
    probs = p / jnp.sum(p)
    probs = jnp.clip(probs, 1e-6, 1.0)

    iota = lax.broadcasted_iota(jnp.int32, (E, 1), 0)
    v1 = jnp.max(probs)
    i1 = jnp.min(jnp.where(probs == v1, iota, E))
    sel1 = iota == i1
    rest = jnp.where(sel1, -jnp.inf, probs)
    v2 = jnp.max(rest)
    i2 = jnp.min(jnp.where((rest == v2) & (~sel1), iota, E))
    sel2 = iota == i2
    denom = v1 + v2 + 1e-8
    wa = v1 / denom
    wb = v2 / denom
    gates = jnp.where(sel1, wa, 0.0) + jnp.where(sel2, wb, 0.0)

    def gate(e):
        return jnp.sum(jnp.where(iota == e, gates, 0.0))

    g0, g1, g2, g3 = gate(0), gate(1), gate(2), gate(3)
    g4, g5, g6, g7 = gate(4), gate(5), gate(6), gate(7)

    def s_vec(fa, fab, fb, fbb):
        h = jnp.maximum(_mm(fa[...], gvec) + fab[...], 0.0)
        return jax.nn.sigmoid(_mm(fb[...], h) + fbb[...])

    alpha = ((wa + wb)
             + g3 * s_vec(f3a_ref, f3ab_ref, f3b_ref, f3bb_ref)
             + g7 * s_vec(f7a_ref, f7ab_ref, f7b_ref, f7bb_ref))

    btot = (g0 * b0_ref[...] + g1 * b1_ref[...] + g2 * b2_ref[...]
            + g4 * b4_ref[...] + g5 * b5_ref[...] + g6 * b6_ref[...])

    out_ref[0] = xb * alpha + btot

    li = lax.broadcasted_iota(jnp.int32, (1, HW), 1)
    m_r = jnp.where(li % W != W - 1, 1.0, 0.0)
    m_l = jnp.where(li % W != 0, 1.0, 0.0)

    def tap(dh, dw):
        s = _lshift(xb, dh * W + dw)
        if dw == 1:
            s = s * m_r
        elif dw == -1:
            s = s * m_l
        return s

    @pl.when(g1 + g5 > 0.0)
    def _freq():
        wfr = g1 * (w1_ref[...] * g1_ref[...]) + g5 * (w5_ref[...] * g5_ref[...])
        out_ref[0] += _mm(wfr, xb)

    @pl.when(g2 + g6 > 0.0)
    def _edge():
        lap = tap(-1, 0) + tap(1, 0) + tap(0, -1) + tap(0, 1) - 4.0 * xb
        wed = g2 * w2_ref[...] + g6 * w6_ref[...]
        out_ref[0] += _mm(wed, lap)

    def texture(dw_ref, db_ref, w_ref, g):
        acc = jnp.broadcast_to(db_ref[...], (DIM, HW))
        for a in range(3):
            for c in range(3):
                acc = acc + tap(a - 1, c - 1) * dw_ref[:, a * 3 + c][:, None]
        u = jax.nn.gelu(acc)
        out_ref[0] += _mm(g * w_ref[...], u)

    @pl.when(g0 > 0.0)
    def _tex0():
        texture(dw0_ref, db0_ref, w0_ref, g0)

    @pl.when(g4 > 0.0)
    def _tex4():
        texture(dw4_ref, db4_ref, w4_ref, g4)


def kernel(x, params):
    B = x.shape[0]
    xf = x.reshape(B, DIM, HW)

    def pw(e):
        return params[f'e{e}_pw_w'].reshape(DIM, DIM)

    def col(v):
        return v[:, None]

    dw0 = params['e0_dw_w'].reshape(DIM, 9)
    dw4 = params['e4_dw_w'].reshape(DIM, 9)

    operands = [
        xf,
        params['router_w'], col(params['router_b']), col(params['expert_bias']),
        pw(0), col(params['e0_pw_b']), dw0, col(params['e0_dw_b']),
        pw(1), col(params['e1_pw_b']), params['e1_gain'][None, :],
        pw(2), col(params['e2_pw_b']),
        params['e3_fc1_w'], col(params['e3_fc1_b']),
        params['e3_fc2_w'], col(params['e3_fc2_b']),
        pw(4), col(params['e4_pw_b']), dw4, col(params['e4_dw_b']),
        pw(5), col(params['e5_pw_b']), params['e5_gain'][None, :],
        pw(6), col(params['e6_pw_b']),
        params['e7_fc1_w'], col(params['e7_fc1_b']),
        params['e7_fc2_w'], col(params['e7_fc2_b']),
    ]

    full = lambda a: pl.BlockSpec(a.shape, lambda b: (0,) * a.ndim)
    in_specs = [pl.BlockSpec((1, DIM, HW), lambda b: (b, 0, 0))]
    in_specs += [full(a) for a in operands[1:]]

    out_f = pl.pallas_call(
        _moe_step,
        grid=(B,),
        in_specs=in_specs,
        out_specs=pl.BlockSpec((1, DIM, HW), lambda b: (b, 0, 0)),
        out_shape=jax.ShapeDtypeStruct((B, DIM, HW), jnp.float32),
    )(*operands)

    out = out_f.reshape(B, DIM, H, W)
    return (out, jnp.array(0.0, dtype=x.dtype))


# speedup vs baseline: 2.0975x; 2.0975x over previous
import jax
import jax.numpy as jnp
from jax import lax
from jax.experimental import pallas as pl

DIM = 192
E = 8
H = 64
W = 64
HW = H * W


def _rshift(a, k):
    if k > 0:
        return jnp.concatenate(
            [a[k:], jnp.zeros((k, a.shape[1]), a.dtype)], axis=0)
    if k < 0:
        return jnp.concatenate(
            [jnp.zeros((-k, a.shape[1]), a.dtype), a[:k]], axis=0)
    return a


def _matmul_ct(a, w):
    return lax.dot_general(a, w, (((1,), (1,)), ((), ())),
                           preferred_element_type=jnp.float32)


def _moe_step(x_ref, rw_ref, rb_ref, eb_ref,
              w0_ref, b0_ref, dw0_ref, db0_ref,
              w1_ref, b1_ref, g1_ref,
              w2_ref, b2_ref,
              f3a_ref, f3ab_ref, f3b_ref, f3bb_ref,
              w4_ref, b4_ref, dw4_ref, db4_ref,
              w5_ref, b5_ref, g5_ref,
              w6_ref, b6_ref,
              f7a_ref, f7ab_ref, f7b_ref, f7bb_ref,
              out_ref):
    xf = x_ref[0]

    gvec = jnp.mean(xf, axis=0, keepdims=True)
    logits = _matmul_ct(gvec, rw_ref[...]) + rb_ref[...]
    logits = jnp.clip(logits, -10.0, 10.0) + eb_ref[...]
    m = jnp.max(logits)
    p = jnp.exp(logits - m)
    probs = p / jnp.sum(p)
    probs = jnp.clip(probs, 1e-6, 1.0)

    iota = lax.broadcasted_iota(jnp.int32, (1, E), 1)
    v1 = jnp.max(probs)
    i1 = jnp.min(jnp.where(probs == v1, iota, E))
    sel1 = iota == i1
    rest = jnp.where(sel1, -jnp.inf, probs)
    v2 = jnp.max(rest)
    i2 = jnp.min(jnp.where((rest == v2) & (~sel1), iota, E))
    sel2 = iota == i2
    denom = v1 + v2 + 1e-8
    wa = v1 / denom
    wb = v2 / denom
    gates = jnp.where(sel1, wa, 0.0) + jnp.where(sel2, wb, 0.0)

    def gate(e):
        return jnp.sum(jnp.where(iota == e, gates, 0.0))

    g0, g1, g2, g3 = gate(0), gate(1), gate(2), gate(3)
    g4, g5, g6, g7 = gate(4), gate(5), gate(6), gate(7)

    def s_vec(fa, fab, fb, fbb):
        h = jnp.maximum(_matmul_ct(gvec, fa[...]) + fab[...], 0.0)
        return jax.nn.sigmoid(_matmul_ct(h, fb[...]) + fbb[...])

    alpha = ((wa + wb)
             + g3 * s_vec(f3a_ref, f3ab_ref, f3b_ref, f3bb_ref)
             + g7 * s_vec(f7a_ref, f7ab_ref, f7b_ref, f7bb_ref))

    btot = (g0 * b0_ref[...] + g1 * b1_ref[...] + g2 * b2_ref[...]
            + g4 * b4_ref[...] + g5 * b5_ref[...] + g6 * b6_ref[...])

    out_ref[0] = xf * alpha + btot

    ri = lax.broadcasted_iota(jnp.int32, (HW, 1), 0)
    m_r = jnp.where(ri % W != W - 1, 1.0, 0.0)
    m_l = jnp.where(ri % W != 0, 1.0, 0.0)

    def tap(dh, dw):
        s = _rshift(xf, dh * W + dw)
        if dw == 1:
            s = s * m_r
        elif dw == -1:
            s = s * m_l
        return s

    @pl.when(g1 + g5 > 0.0)
    def _freq():
        wfr = g1 * (w1_ref[...] * g1_ref[...]) + g5 * (w5_ref[...] * g5_ref[...])
        out_ref[0] += _matmul_ct(xf, wfr)

    @pl.when(g2 + g6 > 0.0)
    def _edge():
        lap = (tap(-1, 0) + tap(1, 0) + tap(0, -1) + tap(0, 1) - 4.0 * xf)
        wed = g2 * w2_ref[...] + g6 * w6_ref[...]
        out_ref[0] += _matmul_ct(lap, wed)

    def texture(dw_ref, db_ref, w_ref, g):
        acc = jnp.broadcast_to(db_ref[...], (HW, DIM))
        for a in range(3):
            for c in range(3):
                acc = acc + tap(a - 1, c - 1) * dw_ref[a * 3 + c][None, :]
        u = jax.nn.gelu(acc)
        out_ref[0] += _matmul_ct(u, g * w_ref[...])

    @pl.when(g0 > 0.0)
    def _tex0():
        texture(dw0_ref, db0_ref, w0_ref, g0)

    @pl.when(g4 > 0.0)
    def _tex4():
        texture(dw4_ref, db4_ref, w4_ref, g4)


def kernel(x, params):
    B = x.shape[0]
    xh = jnp.transpose(x, (0, 2, 3, 1)).reshape(B, HW, DIM)

    def pw(e):
        return params[f'e{e}_pw_w'].reshape(DIM, DIM)

    def row(v):
        return v[None, :]

    dw0 = params['e0_dw_w'].reshape(DIM, 9).T
    dw4 = params['e4_dw_w'].reshape(DIM, 9).T

    operands = [
        xh,
        params['router_w'], row(params['router_b']), row(params['expert_bias']),
        pw(0), row(params['e0_pw_b']), dw0, row(params['e0_dw_b']),
        pw(1), row(params['e1_pw_b']), row(params['e1_gain']),
        pw(2), row(params['e2_pw_b']),
        params['e3_fc1_w'], row(params['e3_fc1_b']),
        params['e3_fc2_w'], row(params['e3_fc2_b']),
        pw(4), row(params['e4_pw_b']), dw4, row(params['e4_dw_b']),
        pw(5), row(params['e5_pw_b']), row(params['e5_gain']),
        pw(6), row(params['e6_pw_b']),
        params['e7_fc1_w'], row(params['e7_fc1_b']),
        params['e7_fc2_w'], row(params['e7_fc2_b']),
    ]

    full = lambda a: pl.BlockSpec(a.shape, lambda b: (0,) * a.ndim)
    in_specs = [pl.BlockSpec((1, HW, DIM), lambda b: (b, 0, 0))]
    in_specs += [full(a) for a in operands[1:]]

    out_h = pl.pallas_call(
        _moe_step,
        grid=(B,),
        in_specs=in_specs,
        out_specs=pl.BlockSpec((1, HW, DIM), lambda b: (b, 0, 0)),
        out_shape=jax.ShapeDtypeStruct((B, HW, DIM), jnp.float32),
    )(*operands)

    out = jnp.transpose(out_h.reshape(B, H, W, DIM), (0, 3, 1, 2))
    return (out, jnp.array(0.0, dtype=x.dtype))
